# baseline (speedup 1.0000x reference)
import functools

import jax
import jax.numpy as jnp
from jax import lax
from jax.experimental import pallas as pl
from jax.experimental.pallas import tpu as pltpu
from jax.experimental.pallas import tpu_sc as plsc

D = 128
N_PAD = 10240
E_PAD = 327680
CHUNK = 80
N_TILES = 32
EDGES_PER_TILE = E_PAD // N_TILES
N_CHUNKS = EDGES_PER_TILE // CHUNK
ROWS_PER_SUB = N_PAD // 16
BN = 2048


def _sc_aggregate(feat_hbm, src_hbm, dst_hbm, w_hbm, cv_hbm,
                  a_out, cnt_out, ew_out,
                  src_v, dst_v, w_v, cv_v, rows_v, scaled_v,
                  a_sh, cnt_sh, ew_sh, isem, gsem0, gsem1, ssem0, ssem1):
    c = lax.axis_index("c")
    s = lax.axis_index("s")
    tid = c * 16 + s
    gsem = (gsem0, gsem1)
    ssem = (ssem0, ssem1)

    def zrow(k, _):
        for j in range(8):
            rows_v[0, k, pl.ds(j * 16, 16)] = jnp.zeros((16,), jnp.float32)
        return 0
    lax.fori_loop(0, CHUNK, zrow, 0)

    roff = s * ROWS_PER_SUB
    zh = []
    for t in range(ROWS_PER_SUB // CHUNK):
        zh.append(pltpu.async_copy(
            rows_v.at[0], a_sh.at[pl.ds(roff + t * CHUNK, CHUNK)], isem))
    for t in range(ROWS_PER_SUB // D):
        zh.append(pltpu.async_copy(
            rows_v.at[0, 0], cnt_sh.at[pl.ds(roff + t * D, D)], isem))
        zh.append(pltpu.async_copy(
            rows_v.at[0, 0], ew_sh.at[pl.ds(roff + t * D, D)], isem))
    for h in zh:
        h.wait()
    plsc.subcore_barrier()

    def idx_start(ci, j):
        eoff = pl.multiple_of(tid * EDGES_PER_TILE + ci * CHUNK, CHUNK)
        pltpu.async_copy(src_hbm.at[pl.ds(eoff, CHUNK)], src_v.at[j], isem)
        pltpu.async_copy(dst_hbm.at[pl.ds(eoff, CHUNK)], dst_v.at[j], isem)
        pltpu.async_copy(w_hbm.at[pl.ds(eoff, CHUNK)], w_v.at[j], isem)
        pltpu.async_copy(cv_hbm.at[pl.ds(eoff, CHUNK)], cv_v.at[j], isem)

    def idx_wait(ci, j):
        eoff = pl.multiple_of(tid * EDGES_PER_TILE + ci * CHUNK, CHUNK)
        pltpu.make_async_copy(src_hbm.at[pl.ds(eoff, CHUNK)], src_v.at[j],
                              isem).wait()
        pltpu.make_async_copy(dst_hbm.at[pl.ds(eoff, CHUNK)], dst_v.at[j],
                              isem).wait()
        pltpu.make_async_copy(w_hbm.at[pl.ds(eoff, CHUNK)], w_v.at[j],
                              isem).wait()
        pltpu.make_async_copy(cv_hbm.at[pl.ds(eoff, CHUNK)], cv_v.at[j],
                              isem).wait()

    def gather_start(j, b):
        pltpu.async_copy(feat_hbm.at[src_v.at[j]], rows_v.at[b], gsem[b])

    def gather_wait(j, b):
        pltpu.make_async_copy(feat_hbm.at[src_v.at[j]], rows_v.at[b],
                              gsem[b]).wait()

    def scatter_start(j, b):
        pltpu.async_copy(scaled_v.at[b], a_sh.at[dst_v.at[j]], ssem[b],
                         add=True)
        pltpu.async_copy(w_v.at[j], ew_sh.at[dst_v.at[j]], ssem[b], add=True)
        pltpu.async_copy(cv_v.at[j], cnt_sh.at[dst_v.at[j]], ssem[b],
                         add=True)

    def scatter_wait(j, b):
        pltpu.make_async_copy(scaled_v.at[b], a_sh.at[dst_v.at[j]],
                              ssem[b]).wait()
        pltpu.make_async_copy(w_v.at[j], ew_sh.at[dst_v.at[j]],
                              ssem[b]).wait()
        pltpu.make_async_copy(cv_v.at[j], cnt_sh.at[dst_v.at[j]],
                              ssem[b]).wait()

    def multiply(j, b):
        def mul_group(g, _):
            wvec = w_v[j, pl.ds(g * 16, 16)]
            for jj in range(16):
                wk = wvec[jj]
                k = g * 16 + jj
                for col in range(8):
                    sl = pl.ds(col * 16, 16)
                    scaled_v[b, k, sl] = rows_v[b, k, sl] * wk
            return 0
        lax.fori_loop(0, CHUNK // 16, mul_group, 0)

    idx_start(0, 0)
    idx_wait(0, 0)
    gather_start(0, 0)
    idx_start(1, 1)

    def chunk_quad(t, _):
        for u in range(4):
            ci = t * 4 + u
            b = u % 2
            ju = u
            jn = (u + 1) % 4
            jn2 = (u + 2) % 4

            @pl.when(ci >= 1)
            def _():
                scatter_wait((u - 1) % 4, 1 - b)

            @pl.when(ci + 1 < N_CHUNKS)
            def _():
                idx_wait(ci + 1, jn)
                gather_start(jn, 1 - b)

            @pl.when(ci + 2 < N_CHUNKS)
            def _():
                idx_start(ci + 2, jn2)

            gather_wait(ju, b)
            multiply(ju, b)
            scatter_start(ju, b)
        return 0
    lax.fori_loop(0, N_CHUNKS // 4, chunk_quad, 0)

    scatter_wait((N_CHUNKS - 1) % 4, 1)
    plsc.subcore_barrier()

    pltpu.sync_copy(a_sh.at[pl.ds(roff, ROWS_PER_SUB)],
                    a_out.at[c, pl.ds(roff, ROWS_PER_SUB)])
    pltpu.sync_copy(cnt_sh.at[pl.ds(roff, ROWS_PER_SUB)],
                    cnt_out.at[c, pl.ds(roff, ROWS_PER_SUB)])
    pltpu.sync_copy(ew_sh.at[pl.ds(roff, ROWS_PER_SUB)],
                    ew_out.at[c, pl.ds(roff, ROWS_PER_SUB)])


_sc_call = functools.partial(
    pl.kernel,
    mesh=plsc.VectorSubcoreMesh(core_axis_name="c", subcore_axis_name="s"),
    out_type=[
        jax.ShapeDtypeStruct((2, N_PAD, D), jnp.float32),
        jax.ShapeDtypeStruct((2, N_PAD), jnp.float32),
        jax.ShapeDtypeStruct((2, N_PAD), jnp.float32),
    ],
    scratch_types=[
        pltpu.VMEM((4, CHUNK), jnp.int32),
        pltpu.VMEM((4, CHUNK), jnp.int32),
        pltpu.VMEM((4, CHUNK), jnp.float32),
        pltpu.VMEM((4, CHUNK), jnp.float32),
        pltpu.VMEM((2, CHUNK, D), jnp.float32),
        pltpu.VMEM((2, CHUNK, D), jnp.float32),
        pltpu.VMEM_SHARED((N_PAD, D), jnp.float32),
        pltpu.VMEM_SHARED((N_PAD,), jnp.float32),
        pltpu.VMEM_SHARED((N_PAD,), jnp.float32),
        pltpu.SemaphoreType.DMA,
        pltpu.SemaphoreType.DMA,
        pltpu.SemaphoreType.DMA,
        pltpu.SemaphoreType.DMA,
        pltpu.SemaphoreType.DMA,
    ],
)(_sc_aggregate)


def _tc_epilogue(a_ref, cnt_ref, ew_ref, feat_ref, alpha_ref, w_ref, b_ref,
                 o_ref):
    a = a_ref[0] + a_ref[1]
    sums = lax.dot_general(a, w_ref[...], (((1,), (1,)), ((), ())),
                           preferred_element_type=jnp.float32)
    cnt = cnt_ref[0, :] + cnt_ref[1, :]
    ew = ew_ref[0, :] + ew_ref[1, :]
    sums = sums + ew[:, None] * b_ref[...][None, :]
    inv = 1.0 / jnp.maximum(cnt, 1.0)
    f = jnp.maximum(sums * inv[:, None], 0.0)
    o_ref[...] = f + alpha_ref[:, 0:1] * feat_ref[...]


_tc_call = pl.pallas_call(
    _tc_epilogue,
    grid=(N_PAD // BN,),
    in_specs=[
        pl.BlockSpec((2, BN, D), lambda i: (0, i, 0)),
        pl.BlockSpec((2, BN), lambda i: (0, i)),
        pl.BlockSpec((2, BN), lambda i: (0, i)),
        pl.BlockSpec((BN, D), lambda i: (i, 0)),
        pl.BlockSpec((BN, 2), lambda i: (i, 0)),
        pl.BlockSpec((D, D), lambda i: (0, 0)),
        pl.BlockSpec((D,), lambda i: (0,)),
    ],
    out_specs=pl.BlockSpec((BN, D), lambda i: (i, 0)),
    out_shape=jax.ShapeDtypeStruct((N_PAD, D), jnp.float32),
)


def kernel(feat, edge_index, edge_e, node_alpha, W, b):
    n, d = feat.shape
    e = edge_index.shape[1]
    pad = E_PAD - e
    src = jnp.concatenate([edge_index[0].astype(jnp.int32),
                           jnp.zeros((pad,), jnp.int32)])
    dst = jnp.concatenate([edge_index[1].astype(jnp.int32),
                           jnp.zeros((pad,), jnp.int32)])
    w = jnp.concatenate([edge_e[:, 0], jnp.zeros((pad,), jnp.float32)])
    cv = jnp.concatenate([jnp.ones((e,), jnp.float32),
                          jnp.zeros((pad,), jnp.float32)])
    feat_p = jnp.pad(feat, ((0, N_PAD - n), (0, 0)))
    alpha_p = jnp.pad(node_alpha, ((0, N_PAD - n), (0, 0)))

    a2, cnt2, ew2 = _sc_call(feat_p, src, dst, w, cv)
    out = _tc_call(a2, cnt2, ew2, feat_p, alpha_p, W, b)
    return out[:n]


# speedup vs baseline: 1.1388x; 1.1388x over previous
import functools

import jax
import jax.numpy as jnp
from jax import lax
from jax.experimental import pallas as pl
from jax.experimental.pallas import tpu as pltpu
from jax.experimental.pallas import tpu_sc as plsc

D = 128
N_PAD = 10240
E_PAD = 327680
CHUNK = 80
N_TILES = 32
EDGES_PER_TILE = E_PAD // N_TILES
N_CHUNKS = EDGES_PER_TILE // CHUNK
ROWS_PER_SUB = N_PAD // 16
BN = 2048


def _sc_aggregate(feat_hbm, src_hbm, dst_hbm, w_hbm, cv_hbm,
                  a_out, cnt_out, ew_out,
                  src_v, dst_v, w_v, cv_v, rows_v, scaled_v,
                  a_sh, cnt_sh, ew_sh, isem, gsem0, gsem1, ssem0, ssem1):
    c = lax.axis_index("c")
    s = lax.axis_index("s")
    tid = c * 16 + s
    gsem = (gsem0, gsem1)
    ssem = (ssem0, ssem1)

    def zrow(k, _):
        for j in range(8):
            scaled_v[0, k, pl.ds(j * 16, 16)] = jnp.zeros((16,), jnp.float32)
        return 0
    lax.fori_loop(0, CHUNK, zrow, 0)

    roff = s * ROWS_PER_SUB
    zh = []
    for t in range(ROWS_PER_SUB // CHUNK):
        zh.append(pltpu.async_copy(
            scaled_v.at[0], a_sh.at[pl.ds(roff + t * CHUNK, CHUNK)], isem))
    for t in range(ROWS_PER_SUB // D):
        zh.append(pltpu.async_copy(
            scaled_v.at[0, 0], cnt_sh.at[pl.ds(roff + t * D, D)], isem))
        zh.append(pltpu.async_copy(
            scaled_v.at[0, 0], ew_sh.at[pl.ds(roff + t * D, D)], isem))
    for h in zh:
        h.wait()
    plsc.subcore_barrier()

    def idx_start(ci, j):
        eoff = pl.multiple_of(tid * EDGES_PER_TILE + ci * CHUNK, CHUNK)
        pltpu.async_copy(src_hbm.at[pl.ds(eoff, CHUNK)], src_v.at[j], isem)
        pltpu.async_copy(dst_hbm.at[pl.ds(eoff, CHUNK)], dst_v.at[j], isem)
        pltpu.async_copy(w_hbm.at[pl.ds(eoff, CHUNK)], w_v.at[j], isem)
        pltpu.async_copy(cv_hbm.at[pl.ds(eoff, CHUNK)], cv_v.at[j], isem)

    def idx_wait(ci, j):
        eoff = pl.multiple_of(tid * EDGES_PER_TILE + ci * CHUNK, CHUNK)
        pltpu.make_async_copy(src_hbm.at[pl.ds(eoff, CHUNK)], src_v.at[j],
                              isem).wait()
        pltpu.make_async_copy(dst_hbm.at[pl.ds(eoff, CHUNK)], dst_v.at[j],
                              isem).wait()
        pltpu.make_async_copy(w_hbm.at[pl.ds(eoff, CHUNK)], w_v.at[j],
                              isem).wait()
        pltpu.make_async_copy(cv_hbm.at[pl.ds(eoff, CHUNK)], cv_v.at[j],
                              isem).wait()

    def gather_start(j, b):
        pltpu.async_copy(feat_hbm.at[src_v.at[j]], rows_v.at[b], gsem[b])

    def gather_wait(j, b):
        pltpu.make_async_copy(feat_hbm.at[src_v.at[j]], rows_v.at[b],
                              gsem[b]).wait()

    def scatter_start(j, b):
        pltpu.async_copy(scaled_v.at[b], a_sh.at[dst_v.at[j]], ssem[b],
                         add=True)
        pltpu.async_copy(w_v.at[j], ew_sh.at[dst_v.at[j]], ssem[b], add=True)
        pltpu.async_copy(cv_v.at[j], cnt_sh.at[dst_v.at[j]], ssem[b],
                         add=True)

    def scatter_wait(j, b):
        pltpu.make_async_copy(scaled_v.at[b], a_sh.at[dst_v.at[j]],
                              ssem[b]).wait()
        pltpu.make_async_copy(w_v.at[j], ew_sh.at[dst_v.at[j]],
                              ssem[b]).wait()
        pltpu.make_async_copy(cv_v.at[j], cnt_sh.at[dst_v.at[j]],
                              ssem[b]).wait()

    def multiply(j, b):
        def mul_group(g, _):
            wvec = w_v[j, pl.ds(g * 16, 16)]
            for jj in range(16):
                wk = wvec[jj]
                k = g * 16 + jj
                for col in range(4):
                    x = rows_v[b, k, pl.ds(col * 16, 16)]
                    lo = lax.bitcast_convert_type(
                        lax.shift_left(x, 16), jnp.float32)
                    hi = lax.bitcast_convert_type(
                        jnp.bitwise_and(x, jnp.int32(-65536)), jnp.float32)
                    scaled_v[b, k, pl.ds(col * 16, 16)] = lo * wk
                    scaled_v[b, k, pl.ds(64 + col * 16, 16)] = hi * wk
            return 0
        lax.fori_loop(0, CHUNK // 16, mul_group, 0)

    idx_start(0, 0)
    idx_wait(0, 0)
    gather_start(0, 0)
    idx_start(1, 1)

    def chunk_quad(t, _):
        for u in range(4):
            ci = t * 4 + u
            b = u % 2
            ju = u
            jn = (u + 1) % 4
            jn2 = (u + 2) % 4

            @pl.when(ci >= 1)
            def _():
                scatter_wait((u - 1) % 4, 1 - b)

            @pl.when(ci + 1 < N_CHUNKS)
            def _():
                idx_wait(ci + 1, jn)
                gather_start(jn, 1 - b)

            @pl.when(ci + 2 < N_CHUNKS)
            def _():
                idx_start(ci + 2, jn2)

            gather_wait(ju, b)
            multiply(ju, b)
            scatter_start(ju, b)
        return 0
    lax.fori_loop(0, N_CHUNKS // 4, chunk_quad, 0)

    scatter_wait((N_CHUNKS - 1) % 4, 1)
    plsc.subcore_barrier()

    pltpu.sync_copy(a_sh.at[pl.ds(roff, ROWS_PER_SUB)],
                    a_out.at[c, pl.ds(roff, ROWS_PER_SUB)])
    pltpu.sync_copy(cnt_sh.at[pl.ds(roff, ROWS_PER_SUB)],
                    cnt_out.at[c, pl.ds(roff, ROWS_PER_SUB)])
    pltpu.sync_copy(ew_sh.at[pl.ds(roff, ROWS_PER_SUB)],
                    ew_out.at[c, pl.ds(roff, ROWS_PER_SUB)])


_sc_call = functools.partial(
    pl.kernel,
    mesh=plsc.VectorSubcoreMesh(core_axis_name="c", subcore_axis_name="s"),
    compiler_params=pltpu.CompilerParams(use_tc_tiling_on_sc=False),
    out_type=[
        jax.ShapeDtypeStruct((2, N_PAD, D), jnp.float32),
        jax.ShapeDtypeStruct((2, N_PAD), jnp.float32),
        jax.ShapeDtypeStruct((2, N_PAD), jnp.float32),
    ],
    scratch_types=[
        pltpu.VMEM((4, CHUNK), jnp.int32),
        pltpu.VMEM((4, CHUNK), jnp.int32),
        pltpu.VMEM((4, CHUNK), jnp.float32),
        pltpu.VMEM((4, CHUNK), jnp.float32),
        pltpu.VMEM((2, CHUNK, D // 2), jnp.int32),
        pltpu.VMEM((2, CHUNK, D), jnp.float32),
        pltpu.VMEM_SHARED((N_PAD, D), jnp.float32),
        pltpu.VMEM_SHARED((N_PAD,), jnp.float32),
        pltpu.VMEM_SHARED((N_PAD,), jnp.float32),
        pltpu.SemaphoreType.DMA,
        pltpu.SemaphoreType.DMA,
        pltpu.SemaphoreType.DMA,
        pltpu.SemaphoreType.DMA,
        pltpu.SemaphoreType.DMA,
    ],
)(_sc_aggregate)


def _tc_epilogue(a_ref, cnt_ref, ew_ref, feat_ref, alpha_ref, w_ref, b_ref,
                 o_ref):
    a = a_ref[0] + a_ref[1]
    sums = lax.dot_general(a, w_ref[...], (((1,), (1,)), ((), ())),
                           preferred_element_type=jnp.float32)
    cnt = cnt_ref[0, :] + cnt_ref[1, :]
    ew = ew_ref[0, :] + ew_ref[1, :]
    sums = sums + ew[:, None] * b_ref[...][None, :]
    inv = 1.0 / jnp.maximum(cnt, 1.0)
    f = jnp.maximum(sums * inv[:, None], 0.0)
    o_ref[...] = f + alpha_ref[:, 0:1] * feat_ref[...]


_tc_call = pl.pallas_call(
    _tc_epilogue,
    grid=(N_PAD // BN,),
    in_specs=[
        pl.BlockSpec((2, BN, D), lambda i: (0, i, 0)),
        pl.BlockSpec((2, BN), lambda i: (0, i)),
        pl.BlockSpec((2, BN), lambda i: (0, i)),
        pl.BlockSpec((BN, D), lambda i: (i, 0)),
        pl.BlockSpec((BN, 2), lambda i: (i, 0)),
        pl.BlockSpec((D, D), lambda i: (0, 0)),
        pl.BlockSpec((D,), lambda i: (0,)),
    ],
    out_specs=pl.BlockSpec((BN, D), lambda i: (i, 0)),
    out_shape=jax.ShapeDtypeStruct((N_PAD, D), jnp.float32),
)


def kernel(feat, edge_index, edge_e, node_alpha, W, b):
    n, d = feat.shape
    e = edge_index.shape[1]
    pad = E_PAD - e
    src = jnp.concatenate([edge_index[0].astype(jnp.int32),
                           jnp.zeros((pad,), jnp.int32)])
    dst = jnp.concatenate([edge_index[1].astype(jnp.int32),
                           jnp.zeros((pad,), jnp.int32)])
    w = jnp.concatenate([edge_e[:, 0], jnp.zeros((pad,), jnp.float32)])
    cv = jnp.concatenate([jnp.ones((e,), jnp.float32),
                          jnp.zeros((pad,), jnp.float32)])
    feat_p = jnp.pad(feat, ((0, N_PAD - n), (0, 0)))
    alpha_p = jnp.pad(node_alpha, ((0, N_PAD - n), (0, 0)))

    perm = jnp.stack([jnp.arange(64), 64 + jnp.arange(64)],
                     axis=1).reshape(-1)
    feat_shuf = feat_p[:, perm].astype(jnp.bfloat16)
    feat_i32 = jax.lax.bitcast_convert_type(
        feat_shuf.reshape(N_PAD, D // 2, 2), jnp.int32)
    a2, cnt2, ew2 = _sc_call(feat_i32, src, dst, w, cv)
    out = _tc_call(a2, cnt2, ew2, feat_p, alpha_p, W, b)
    return out[:n]
